# baseline (speedup 1.0000x reference)
import functools

import jax
import jax.numpy as jnp
from jax import lax
from jax.experimental import pallas as pl
from jax.experimental.pallas import tpu as pltpu
from jax.experimental.pallas import tpu_sc as plsc

EMBED = 64
NC = 2
NS = 16
NW = NC * NS
CHUNK = 128


def _make_gather(n_total):
    per_w = n_total // NW
    nch = per_w // CHUNK
    mesh = plsc.VectorSubcoreMesh(core_axis_name="c", subcore_axis_name="s")

    @functools.partial(
        pl.kernel,
        mesh=mesh,
        out_type=jax.ShapeDtypeStruct((n_total, EMBED), jnp.float32),
        compiler_params=pltpu.CompilerParams(use_tc_tiling_on_sc=False),
        scratch_types=[
            pltpu.VMEM((nch, CHUNK), jnp.int32),
            pltpu.VMEM((CHUNK, EMBED), jnp.float32),
            pltpu.SemaphoreType.DMA,
        ],
    )
    def gather_kernel(table_hbm, idx_hbm, out_hbm, idx_v, rows_v, sem):
        wid = lax.axis_index("s") * NC + lax.axis_index("c")
        pltpu.sync_copy(idx_hbm.at[pl.ds(wid * nch, nch)], idx_v)

        def body(j, carry):
            pltpu.async_copy(table_hbm.at[idx_v.at[j]], rows_v, sem).wait()
            pltpu.sync_copy(
                rows_v, out_hbm.at[pl.ds(wid * per_w + j * CHUNK, CHUNK)]
            )
            return carry

        lax.fori_loop(0, nch, body, 0)

    return gather_kernel


def kernel(input_x, table):
    batch, seq = input_x.shape
    n = batch * seq
    idx = input_x.reshape(n // CHUNK, CHUNK).astype(jnp.int32)
    out = _make_gather(n)(table, idx)
    return out.reshape(batch, seq, EMBED)


# speedup vs baseline: 1.1180x; 1.1180x over previous
import functools

import jax
import jax.numpy as jnp
from jax import lax
from jax.experimental import pallas as pl
from jax.experimental.pallas import tpu as pltpu
from jax.experimental.pallas import tpu_sc as plsc

EMBED = 64
NC = 2
NS = 16
NW = NC * NS
CHUNK = 128
NBUF = 8
LOOK = 4


def _make_gather(n_total):
    per_w = n_total // NW
    nch = per_w // CHUNK
    ngroups = nch // NBUF
    assert nch % NBUF == 0 and ngroups >= 3
    mesh = plsc.VectorSubcoreMesh(core_axis_name="c", subcore_axis_name="s")

    @functools.partial(
        pl.kernel,
        mesh=mesh,
        out_type=jax.ShapeDtypeStruct((n_total, EMBED), jnp.float32),
        compiler_params=pltpu.CompilerParams(use_tc_tiling_on_sc=False),
        scratch_types=[
            pltpu.VMEM((nch, CHUNK), jnp.int32),
            pltpu.VMEM((NBUF, CHUNK, EMBED), jnp.float32),
            pltpu.SemaphoreType.DMA((NBUF,)),
            pltpu.SemaphoreType.DMA((NBUF,)),
        ],
    )
    def gather_kernel(table_hbm, idx_hbm, out_hbm, idx_v, rows_v, gsem, wsem):
        wid = lax.axis_index("s") * NC + lax.axis_index("c")
        base = wid * per_w
        pltpu.sync_copy(idx_hbm.at[pl.ds(wid * nch, nch)], idx_v)

        def gather_start(j, b):
            pltpu.async_copy(table_hbm.at[idx_v.at[j]], rows_v.at[b], gsem.at[b])

        def gather_wait(j, b):
            pltpu.make_async_copy(
                table_hbm.at[idx_v.at[j]], rows_v.at[b], gsem.at[b]
            ).wait()

        def wb_start(j, b):
            pltpu.async_copy(
                rows_v.at[b],
                out_hbm.at[pl.ds(base + j * CHUNK, CHUNK)],
                wsem.at[b],
            )

        def wb_wait(j, b):
            pltpu.make_async_copy(
                rows_v.at[b],
                out_hbm.at[pl.ds(base + j * CHUNK, CHUNK)],
                wsem.at[b],
            ).wait()

        for j in range(LOOK):
            gather_start(j, j % NBUF)

        def step(j, b, bn, first_group):
            jn = j + LOOK
            if not (first_group and jn < NBUF):
                wb_wait(jn - NBUF, bn)
            gather_start(jn, bn)
            gather_wait(j, b)
            wb_start(j, b)

        for b in range(NBUF):
            step(b, b, (b + LOOK) % NBUF, True)

        def group(g, carry):
            j0 = g * NBUF
            for b in range(NBUF):
                step(j0 + b, b, (b + LOOK) % NBUF, False)
            return carry

        lax.fori_loop(1, ngroups - 1, group, 0)

        j0 = (ngroups - 1) * NBUF
        for b in range(NBUF):
            j = j0 + b
            jn = j + LOOK
            bn = (b + LOOK) % NBUF
            if jn < nch:
                wb_wait(jn - NBUF, bn)
                gather_start(jn, bn)
            gather_wait(j, b)
            wb_start(j, b)

        for j in range(nch - NBUF, nch):
            wb_wait(j, j % NBUF)

    return gather_kernel


def kernel(input_x, table):
    batch, seq = input_x.shape
    n = batch * seq
    idx = input_x.reshape(n // CHUNK, CHUNK).astype(jnp.int32)
    out = _make_gather(n)(table, idx)
    return out.reshape(batch, seq, EMBED)
